# baseline (speedup 1.0000x reference)
import functools

import jax
import jax.numpy as jnp
from jax import lax
from jax.experimental import pallas as pl
from jax.experimental.pallas import tpu as pltpu
from jax.experimental.pallas import tpu_sc as plsc

L = 16


def _make_sc_kernel(BN, Nn, Bn, D, CH):
    info = plsc.get_sparse_core_info()
    NC, NS = info.num_cores, info.num_subcores
    NW = NC * NS
    pos_per_w = Nn // NW
    n_chunks = pos_per_w // CH
    mesh = plsc.VectorSubcoreMesh(core_axis_name="c", subcore_axis_name="s")

    @functools.partial(
        pl.kernel,
        mesh=mesh,
        out_type=jax.ShapeDtypeStruct((BN, D), jnp.float32),
        scratch_types=[
            pltpu.VMEM((Bn, pos_per_w), jnp.int32),
            pltpu.VMEM((CH, D), jnp.float32),
            pltpu.VMEM((CH, D), jnp.float32),
            pltpu.SemaphoreType.DMA,
        ],
    )
    def k(table_hbm, tok_hbm, pos_hbm, out_hbm, idx_v, pbuf, gbuf, sem):
        wid = lax.axis_index("s") * NC + lax.axis_index("c")
        nbase = wid * pos_per_w

        for b in range(Bn):
            pltpu.sync_copy(tok_hbm.at[pl.ds(b * Nn + nbase, pos_per_w)],
                            idx_v.at[b])

        def chunk_body(c, _):
            cbase = nbase + c * CH
            pltpu.sync_copy(pos_hbm.at[pl.ds(cbase, CH)], pbuf)
            for b in range(Bn):
                idx = idx_v.at[b, pl.ds(c * CH, CH)]
                pltpu.async_copy(table_hbm.at[idx], gbuf, sem).wait()

                def row_body(r, _):
                    for j in range(D // L):
                        sl = pl.ds(j * L, L)
                        plsc.addupdate(gbuf.at[r, sl], pbuf[r, sl])
                    return 0

                lax.fori_loop(0, CH, row_body, 0)
                pltpu.sync_copy(gbuf, out_hbm.at[pl.ds(b * Nn + cbase, CH)])
            return 0

        lax.fori_loop(0, n_chunks, chunk_body, 0)

    return k


def kernel(byte_tokens, byte_embeddings, pos_embedding):
    Bn, Nn = byte_tokens.shape
    D = byte_embeddings.shape[1]
    tok_flat = byte_tokens.reshape(-1).astype(jnp.int32)
    pos_flat = pos_embedding[0, :Nn]
    k = _make_sc_kernel(Bn * Nn, Nn, Bn, D, CH=32)
    out = k(byte_embeddings, tok_flat, pos_flat)
    return out.reshape(Bn, Nn, D)


# speedup vs baseline: 1.8446x; 1.8446x over previous
import functools

import jax
import jax.numpy as jnp
from jax import lax
from jax.experimental import pallas as pl
from jax.experimental.pallas import tpu as pltpu
from jax.experimental.pallas import tpu_sc as plsc

L = 16


def _make_sc_kernel(BN, Nn, Bn, D, CH):
    info = plsc.get_sparse_core_info()
    NC, NS = info.num_cores, info.num_subcores
    NW = NC * NS
    pos_per_w = Nn // NW
    n_chunks = pos_per_w // CH
    n_steps = n_chunks * Bn
    DV = D // L
    LB = Bn.bit_length() - 1
    LDV = DV.bit_length() - 1
    assert (1 << LB) == Bn and (1 << LDV) == DV
    mesh = plsc.VectorSubcoreMesh(core_axis_name="c", subcore_axis_name="s")

    @functools.partial(
        pl.kernel,
        mesh=mesh,
        out_type=jax.ShapeDtypeStruct((BN, D), jnp.float32),
        scratch_types=[
            pltpu.VMEM((Bn, pos_per_w), jnp.int32),
            pltpu.VMEM((2, CH, D), jnp.float32),
            pltpu.VMEM((2, CH, D), jnp.float32),
            pltpu.SemaphoreType.DMA,
            pltpu.SemaphoreType.DMA,
            pltpu.SemaphoreType.DMA,
        ],
    )
    def k(table_hbm, tok_hbm, pos_hbm, out_hbm, idx_v, pbuf, gbuf, gsem, osem, psem):
        wid = lax.axis_index("s") * NC + lax.axis_index("c")
        nbase = wid * pos_per_w

        for b in range(Bn):
            pltpu.sync_copy(tok_hbm.at[pl.ds(b * Nn + nbase, pos_per_w)],
                            idx_v.at[b])

        def gather_desc(s):
            c = lax.shift_right_logical(s, LB)
            b = s & (Bn - 1)
            idx = idx_v.at[b, pl.ds(c * CH, CH)]
            return pltpu.make_async_copy(table_hbm.at[idx], gbuf.at[s & 1], gsem)

        def out_desc(s):
            c = lax.shift_right_logical(s, LB)
            b = s & (Bn - 1)
            dst = out_hbm.at[pl.ds(b * Nn + nbase + c * CH, CH)]
            return pltpu.make_async_copy(gbuf.at[s & 1], dst, osem)

        def pos_desc(c):
            src = pos_hbm.at[pl.ds(nbase + c * CH, CH)]
            return pltpu.make_async_copy(src, pbuf.at[c & 1], psem)

        pos_desc(0).start()
        gather_desc(0).start()

        def step(s, _):
            par = s & 1
            c = lax.shift_right_logical(s, LB)
            gather_desc(s).wait()

            @pl.when((s & (Bn - 1)) == 0)
            def _():
                pos_desc(c).wait()

                @pl.when(c + 1 < n_chunks)
                def _():
                    pos_desc(c + 1).start()

            pc = c & 1

            @plsc.parallel_loop(0, CH * DV, unroll=8)
            def add_body(i):
                r = lax.shift_right_logical(i, LDV)
                col = lax.shift_left(i & (DV - 1), 4)
                sl = pl.ds(pl.multiple_of(col, L), L)
                plsc.addupdate(gbuf.at[par, r, sl], pbuf[pc, r, sl])

            out_desc(s).start()

            @pl.when(s + 1 < n_steps)
            def _():
                @pl.when(s >= 1)
                def _():
                    out_desc(s - 1).wait()

                gather_desc(s + 1).start()

            return 0

        lax.fori_loop(0, n_steps, step, 0)
        out_desc(n_steps - 2).wait()
        out_desc(n_steps - 1).wait()

    return k


def kernel(byte_tokens, byte_embeddings, pos_embedding):
    Bn, Nn = byte_tokens.shape
    D = byte_embeddings.shape[1]
    tok_flat = byte_tokens.reshape(-1).astype(jnp.int32)
    pos_flat = pos_embedding[0, :Nn]
    k = _make_sc_kernel(Bn * Nn, Nn, Bn, D, CH=16)
    out = k(byte_embeddings, tok_flat, pos_flat)
    return out.reshape(Bn, Nn, D)
